# baseline (speedup 1.0000x reference)
import jax
import jax.numpy as jnp
from jax.experimental import pallas as pl
from jax.experimental.pallas import tpu as pltpu

_NBUF = 8
_CHUNK_ROWS = 512


def _dma_ring(x_hbm, o_hbm, bufs, load_sems, store_sems):
    nchunks = x_hbm.shape[0] // _CHUNK_ROWS

    def load(i, b):
        return pltpu.make_async_copy(
            x_hbm.at[pl.ds(i * _CHUNK_ROWS, _CHUNK_ROWS), :],
            bufs.at[b],
            load_sems.at[b],
        )

    def store(i, b):
        return pltpu.make_async_copy(
            bufs.at[b],
            o_hbm.at[pl.ds(i * _CHUNK_ROWS, _CHUNK_ROWS), :],
            store_sems.at[b],
        )

    for i in range(min(_NBUF, nchunks)):
        load(i, i).start()
    for i in range(nchunks):
        b = i % _NBUF
        load(i, b).wait()
        store(i, b).start()
        nxt = i + _NBUF
        if nxt < nchunks:
            store(nxt - _NBUF, b).wait()
            load(nxt, b).start()
    for i in range(max(nchunks - _NBUF, 0), nchunks):
        store(i, i % _NBUF).wait()


def kernel(x):
    rows, cols = x.shape
    gathered = pl.pallas_call(
        _dma_ring,
        in_specs=[pl.BlockSpec(memory_space=pl.ANY)],
        out_specs=pl.BlockSpec(memory_space=pl.ANY),
        out_shape=jax.ShapeDtypeStruct((rows, cols), x.dtype),
        scratch_shapes=[
            pltpu.VMEM((_NBUF, _CHUNK_ROWS, cols), x.dtype),
            pltpu.SemaphoreType.DMA((_NBUF,)),
            pltpu.SemaphoreType.DMA((_NBUF,)),
        ],
    )(x)
    sizes = jnp.array([rows], dtype=jnp.int64)
    return (gathered, sizes)


# speedup vs baseline: 1.0142x; 1.0142x over previous
import jax
import jax.numpy as jnp
from jax.experimental import pallas as pl
from jax.experimental.pallas import tpu as pltpu

_NBUF = 3
_CHUNK_ROWS = 2048


def _dma_ring(x_hbm, o_hbm, bufs, load_sems, store_sems):
    nchunks = x_hbm.shape[0] // _CHUNK_ROWS

    def load(i, b):
        return pltpu.make_async_copy(
            x_hbm.at[pl.ds(i * _CHUNK_ROWS, _CHUNK_ROWS), :],
            bufs.at[b],
            load_sems.at[b],
        )

    def store(i, b):
        return pltpu.make_async_copy(
            bufs.at[b],
            o_hbm.at[pl.ds(i * _CHUNK_ROWS, _CHUNK_ROWS), :],
            store_sems.at[b],
        )

    for i in range(min(_NBUF, nchunks)):
        load(i, i).start()
    for i in range(nchunks):
        b = i % _NBUF
        load(i, b).wait()
        store(i, b).start()
        nxt = i + _NBUF
        if nxt < nchunks:
            store(nxt - _NBUF, b).wait()
            load(nxt, b).start()
    for i in range(max(nchunks - _NBUF, 0), nchunks):
        store(i, i % _NBUF).wait()


def kernel(x):
    rows, cols = x.shape
    gathered = pl.pallas_call(
        _dma_ring,
        in_specs=[pl.BlockSpec(memory_space=pl.ANY)],
        out_specs=pl.BlockSpec(memory_space=pl.ANY),
        out_shape=jax.ShapeDtypeStruct((rows, cols), x.dtype),
        scratch_shapes=[
            pltpu.VMEM((_NBUF, _CHUNK_ROWS, cols), x.dtype),
            pltpu.SemaphoreType.DMA((_NBUF,)),
            pltpu.SemaphoreType.DMA((_NBUF,)),
        ],
    )(x)
    sizes = jnp.array([rows], dtype=jnp.int64)
    return (gathered, sizes)
